# Initial kernel scaffold; baseline (speedup 1.0000x reference)
import functools

import jax
import jax.numpy as jnp
from jax import lax
from jax.experimental import pallas as pl
from jax.experimental.pallas import tpu as pltpu
from jax.experimental.pallas import tpu_sc as plsc

_INFO = plsc.get_sparse_core_info()
_NC = _INFO.num_cores
_NS = _INFO.num_subcores
_NW = _NC * _NS

_CHUNK = 1024


@functools.partial(jax.jit, static_argnums=(2, 3))
def _sc_gather(idx, table, n, d):
    per_w = n // _NW
    nsteps = per_w // _CHUNK
    mesh = plsc.VectorSubcoreMesh(core_axis_name="c", subcore_axis_name="s")

    @functools.partial(
        pl.kernel,
        mesh=mesh,
        out_type=jax.ShapeDtypeStruct((n, d), jnp.float32),
        scratch_types=[
            pltpu.VMEM((_CHUNK,), jnp.int32),
            pltpu.VMEM((_CHUNK, d), jnp.float32),
            pltpu.SemaphoreType.DMA,
        ],
    )
    def k(idx_hbm, tab_hbm, out_hbm, idx_v, rows_v, sem):
        wid = lax.axis_index("s") * _NC + lax.axis_index("c")
        base = wid * per_w

        def step(i, carry):
            off = base + i * _CHUNK
            pltpu.sync_copy(idx_hbm.at[pl.ds(off, _CHUNK)], idx_v)
            pltpu.async_copy(tab_hbm.at[idx_v], rows_v, sem).wait()
            pltpu.sync_copy(rows_v, out_hbm.at[pl.ds(off, _CHUNK)])
            return carry

        lax.fori_loop(0, nsteps, step, 0)

    return k(idx, table)


def kernel(x, table):
    b, h = x.shape
    v, d = table.shape
    n = b * h
    out = _sc_gather(x.reshape(n), table, n, d)
    return out.reshape(b, h, d)


# speedup vs baseline: 4.8094x; 4.8094x over previous
import functools

import jax
import jax.numpy as jnp
from jax import lax
from jax.experimental import pallas as pl
from jax.experimental.pallas import tpu as pltpu
from jax.experimental.pallas import tpu_sc as plsc

_INFO = plsc.get_sparse_core_info()
_NC = _INFO.num_cores
_NS = _INFO.num_subcores
_NW = _NC * _NS

_CHUNK = 1024


@functools.partial(jax.jit, static_argnums=(2, 3))
def _sc_gather(idx, table, n, d):
    per_w = n // _NW
    nsteps = per_w // _CHUNK
    mesh = plsc.VectorSubcoreMesh(core_axis_name="c", subcore_axis_name="s")

    @functools.partial(
        pl.kernel,
        mesh=mesh,
        out_type=jax.ShapeDtypeStruct((n, d), jnp.float32),
        scratch_types=[
            pltpu.VMEM((_CHUNK,), jnp.int32),
            pltpu.VMEM((_CHUNK, d), jnp.float32),
            pltpu.SemaphoreType.DMA,
        ],
        compiler_params=pltpu.CompilerParams(use_tc_tiling_on_sc=False),
    )
    def k(idx_hbm, tab_hbm, out_hbm, idx_v, rows_v, sem):
        wid = lax.axis_index("s") * _NC + lax.axis_index("c")
        base = wid * per_w

        def step(i, carry):
            off = base + i * _CHUNK
            pltpu.sync_copy(idx_hbm.at[pl.ds(off, _CHUNK)], idx_v)
            pltpu.async_copy(tab_hbm.at[idx_v], rows_v, sem).wait()
            pltpu.sync_copy(rows_v, out_hbm.at[pl.ds(off, _CHUNK)])
            return carry

        lax.fori_loop(0, nsteps, step, 0)

    return k(idx, table)


def kernel(x, table):
    b, h = x.shape
    v, d = table.shape
    n = b * h
    out = _sc_gather(x.reshape(n), table, n, d)
    return out.reshape(b, h, d)


# speedup vs baseline: 5.0522x; 1.0505x over previous
import functools

import jax
import jax.numpy as jnp
from jax import lax
from jax.experimental import pallas as pl
from jax.experimental.pallas import tpu as pltpu
from jax.experimental.pallas import tpu_sc as plsc

_INFO = plsc.get_sparse_core_info()
_NC = _INFO.num_cores
_NS = _INFO.num_subcores
_NW = _NC * _NS

_CHUNK = 800
_NBUF = 4


@functools.partial(jax.jit, static_argnums=(2, 3))
def _sc_gather(idx, table, n, d):
    per_w = n // _NW
    nsteps = per_w // _CHUNK
    ngroups = nsteps // _NBUF
    assert per_w % _CHUNK == 0 and nsteps % _NBUF == 0 and ngroups >= 2
    mesh = plsc.VectorSubcoreMesh(core_axis_name="c", subcore_axis_name="s")

    @functools.partial(
        pl.kernel,
        mesh=mesh,
        out_type=jax.ShapeDtypeStruct((n, d), jnp.float32),
        scratch_types=(
            [pltpu.VMEM((_CHUNK,), jnp.int32) for _ in range(_NBUF)]
            + [pltpu.VMEM((_CHUNK, d), jnp.float32) for _ in range(_NBUF)]
            + [pltpu.SemaphoreType.DMA for _ in range(3 * _NBUF)]
        ),
        compiler_params=pltpu.CompilerParams(use_tc_tiling_on_sc=False),
    )
    def k(idx_hbm, tab_hbm, out_hbm, *scratch):
        idx_v = scratch[:_NBUF]
        rows_v = scratch[_NBUF:2 * _NBUF]
        si = scratch[2 * _NBUF:3 * _NBUF]
        sg = scratch[3 * _NBUF:4 * _NBUF]
        so = scratch[4 * _NBUF:5 * _NBUF]

        wid = lax.axis_index("s") * _NC + lax.axis_index("c")
        base = wid * per_w

        def idx_copy(i, b):
            return pltpu.make_async_copy(
                idx_hbm.at[pl.ds(base + i * _CHUNK, _CHUNK)], idx_v[b], si[b])

        def gather_copy(b):
            return pltpu.make_async_copy(tab_hbm.at[idx_v[b]], rows_v[b], sg[b])

        def out_copy(i, b):
            return pltpu.make_async_copy(
                rows_v[b], out_hbm.at[pl.ds(base + i * _CHUNK, _CHUNK)], so[b])

        def do_step(i, b, wait_prev_out, prefetch, start_next_gather):
            gather_copy(b).wait()
            out_copy(i, b).start()
            if prefetch:
                idx_copy(i + _NBUF, b).start()
            if start_next_gather:
                b2 = (b + 2) % _NBUF
                idx_copy(0, b2).wait()
                if wait_prev_out:
                    out_copy(0, b2).wait()
                gather_copy(b2).start()

        for b in range(_NBUF):
            idx_copy(b, b).start()
        idx_copy(0, 0).wait()
        gather_copy(0).start()
        idx_copy(0, 1).wait()
        gather_copy(1).start()

        for b in range(_NBUF):
            do_step(b, b, wait_prev_out=(b >= 2), prefetch=True,
                    start_next_gather=True)

        def group(g, carry):
            i0 = g * _NBUF
            for b in range(_NBUF):
                do_step(i0 + b, b, True, True, True)
            return carry

        lax.fori_loop(1, ngroups - 1, group, 0)

        i0 = (ngroups - 1) * _NBUF
        for b in range(_NBUF):
            do_step(i0 + b, b, wait_prev_out=True, prefetch=False,
                    start_next_gather=(b < 2))

        for b in range(_NBUF):
            out_copy(0, b).wait()

    return k(idx, table)


def kernel(x, table):
    b, h = x.shape
    v, d = table.shape
    n = b * h
    out = _sc_gather(x.reshape(n), table, n, d)
    return out.reshape(b, h, d)


# speedup vs baseline: 5.0531x; 1.0002x over previous
import functools

import jax
import jax.numpy as jnp
from jax import lax
from jax.experimental import pallas as pl
from jax.experimental.pallas import tpu as pltpu
from jax.experimental.pallas import tpu_sc as plsc

_INFO = plsc.get_sparse_core_info()
_NC = _INFO.num_cores
_NS = _INFO.num_subcores
_NW = _NC * _NS

_CHUNK = 800
_NBUF = 4


@functools.partial(jax.jit, static_argnums=(2, 3, 4))
def _sc_gather(idx, table, bsz, h, d):
    n = bsz * h
    per_w = n // _NW
    nsteps = per_w // _CHUNK
    ngroups = nsteps // _NBUF
    rps = _CHUNK // h
    assert per_w % _CHUNK == 0 and nsteps % _NBUF == 0 and ngroups >= 2
    assert _CHUNK % h == 0
    mesh = plsc.VectorSubcoreMesh(core_axis_name="c", subcore_axis_name="s")

    @functools.partial(
        pl.kernel,
        mesh=mesh,
        out_type=jax.ShapeDtypeStruct((bsz, h, d), jnp.float32),
        scratch_types=(
            [pltpu.VMEM((_CHUNK,), jnp.int32) for _ in range(_NBUF)]
            + [pltpu.VMEM((_CHUNK, d), jnp.float32) for _ in range(_NBUF)]
            + [pltpu.SemaphoreType.DMA for _ in range(3 * _NBUF)]
        ),
        compiler_params=pltpu.CompilerParams(use_tc_tiling_on_sc=False),
    )
    def k(idx_hbm, tab_hbm, out_hbm, *scratch):
        idx_v = scratch[:_NBUF]
        rows_v = scratch[_NBUF:2 * _NBUF]
        si = scratch[2 * _NBUF:3 * _NBUF]
        sg = scratch[3 * _NBUF:4 * _NBUF]
        so = scratch[4 * _NBUF:5 * _NBUF]

        wid = lax.axis_index("s") * _NC + lax.axis_index("c")
        base = wid * per_w
        base_row = wid * (per_w // h)

        def idx_copy(i, b):
            return pltpu.make_async_copy(
                idx_hbm.at[pl.ds(base + i * _CHUNK, _CHUNK)], idx_v[b], si[b])

        def gather_copy(b):
            return pltpu.make_async_copy(tab_hbm.at[idx_v[b]], rows_v[b], sg[b])

        def out_copy(i, b, j):
            return pltpu.make_async_copy(
                rows_v[b].at[pl.ds(j * h, h)],
                out_hbm.at[base_row + i * rps + j], so[b])

        def out_start(i, b):
            for j in range(rps):
                out_copy(i, b, j).start()

        def out_wait(b):
            for j in range(rps):
                out_copy(0, b, j).wait()

        def do_step(i, b, wait_prev_out, prefetch, start_next_gather):
            gather_copy(b).wait()
            out_start(i, b)
            if prefetch:
                idx_copy(i + _NBUF, b).start()
            if start_next_gather:
                b2 = (b + 2) % _NBUF
                idx_copy(0, b2).wait()
                if wait_prev_out:
                    out_wait(b2)
                gather_copy(b2).start()

        for b in range(_NBUF):
            idx_copy(b, b).start()
        idx_copy(0, 0).wait()
        gather_copy(0).start()
        idx_copy(0, 1).wait()
        gather_copy(1).start()

        for b in range(_NBUF):
            do_step(b, b, wait_prev_out=(b >= 2), prefetch=True,
                    start_next_gather=True)

        def group(g, carry):
            i0 = g * _NBUF
            for b in range(_NBUF):
                do_step(i0 + b, b, True, True, True)
            return carry

        lax.fori_loop(1, ngroups - 1, group, 0)

        i0 = (ngroups - 1) * _NBUF
        for b in range(_NBUF):
            do_step(i0 + b, b, wait_prev_out=True, prefetch=False,
                    start_next_gather=(b < 2))

        for b in range(_NBUF):
            out_wait(b)

    return k(idx, table)


def kernel(x, table):
    b, h = x.shape
    v, d = table.shape
    return _sc_gather(x.reshape(b * h), table, b, h, d)
